# baseline (speedup 1.0000x reference)
import jax
import jax.numpy as jnp
from jax import lax
from jax.experimental import pallas as pl
from jax.experimental.pallas import tpu as pltpu
from jax.experimental.pallas import tpu_sc as plsc

_N = 10000
_E = 320000
_NC = 2
_NS = 16
_NW = _NC * _NS
_EPW = _E // _NW
_CH = 128
_NCHUNK = 80
_EPW_PAD = _NCHUNK * _CH
_NPAIR = _NCHUNK // 2
_BLK = 8
_NBLK = _NCHUNK // _BLK
_NP = _N + 8
_RPT = 624
_RPT_LAST = _N - (_NS - 1) * _RPT

_mesh = plsc.VectorSubcoreMesh(core_axis_name="c", subcore_axis_name="s")


def _per_tile_rows(s, fn):
    @pl.when(s < _NS - 1)
    def _():
        fn(s * _RPT, _RPT)

    @pl.when(s == _NS - 1)
    def _():
        fn((_NS - 1) * _RPT, _RPT_LAST)


def _deg_body(dstw_hbm, ones_hbm, onesc_hbm, out_hbm, dblk, ones_v, tmp, sem):
    del sem
    c = lax.axis_index("c")
    s = lax.axis_index("s")
    wid = c * _NS + s
    _per_tile_rows(s, lambda r0, nr: pltpu.sync_copy(
        ones_hbm.at[pl.ds(r0, nr)], tmp.at[pl.ds(r0, nr)]))
    pltpu.sync_copy(onesc_hbm, ones_v)
    plsc.subcore_barrier()

    def block(b, carry):
        pltpu.sync_copy(dstw_hbm.at[wid, pl.ds(b * _BLK, _BLK)], dblk)
        for k in range(_BLK):
            pltpu.sync_copy(ones_v, tmp.at[dblk.at[k]], add=True)
        return carry

    lax.fori_loop(0, _NBLK, block, 0)
    plsc.subcore_barrier()
    _per_tile_rows(s, lambda r0, nr: pltpu.sync_copy(
        tmp.at[pl.ds(r0, nr)], out_hbm.at[c, pl.ds(r0, nr)]))


_deg_call = pl.kernel(
    _deg_body,
    out_type=jax.ShapeDtypeStruct((_NC, _N, 16), jnp.float32),
    mesh=_mesh,
    scratch_types=[
        pltpu.VMEM((_BLK, _CH), jnp.int32),
        pltpu.VMEM((_CH, 16), jnp.float32),
        pltpu.VMEM_SHARED((_NP, 16), jnp.float32),
        pltpu.SemaphoreType.DMA,
    ],
)


_SCH = 80
_SNCH = _EPW_PAD // _SCH
_SQ = _SNCH // 2


def _scat_body(u_hbm, src_hbm, dst_hbm, out_hbm,
               srcv0, srcv1, dstv0, dstv1, rows0, rows1, tmp, sem0, sem1):
    c = lax.axis_index("c")
    s = lax.axis_index("s")
    wid = c * _NS + s
    _per_tile_rows(s, lambda r0, nr: pltpu.sync_copy(
        u_hbm.at[pl.ds(r0, nr)], tmp.at[pl.ds(r0, nr)]))
    plsc.subcore_barrier()
    base = wid * _EPW_PAD

    def pair(p, carry):
        e0 = pl.ds(base + (2 * p) * _SCH, _SCH)
        e1 = pl.ds(base + (2 * p + 1) * _SCH, _SCH)
        pltpu.sync_copy(src_hbm.at[e0], srcv0)
        pltpu.sync_copy(dst_hbm.at[e0], dstv0)
        pltpu.sync_copy(src_hbm.at[e1], srcv1)
        pltpu.sync_copy(dst_hbm.at[e1], dstv1)
        d0 = pltpu.async_copy(u_hbm.at[srcv0], rows0, sem0)
        d1 = pltpu.async_copy(u_hbm.at[srcv1], rows1, sem1)
        d0.wait()
        pltpu.sync_copy(rows0, tmp.at[dstv0], add=True)
        d1.wait()
        pltpu.sync_copy(rows1, tmp.at[dstv1], add=True)
        return carry

    lax.fori_loop(0, _SQ, pair, 0)
    plsc.subcore_barrier()
    _per_tile_rows(s, lambda r0, nr: pltpu.sync_copy(
        tmp.at[pl.ds(r0, nr)], out_hbm.at[c, pl.ds(r0, nr)]))


_scat_call = pl.kernel(
    _scat_body,
    out_type=jax.ShapeDtypeStruct((_NC, _N, 128), jnp.float32),
    mesh=_mesh,
    scratch_types=[
        pltpu.VMEM((_SCH,), jnp.int32),
        pltpu.VMEM((_SCH,), jnp.int32),
        pltpu.VMEM((_SCH,), jnp.int32),
        pltpu.VMEM((_SCH,), jnp.int32),
        pltpu.VMEM((_SCH, 128), jnp.float32),
        pltpu.VMEM((_SCH, 128), jnp.float32),
        pltpu.VMEM_SHARED((_NP, 128), jnp.float32),
        pltpu.SemaphoreType.DMA,
        pltpu.SemaphoreType.DMA,
    ],
)


def _elu(v):
    return jnp.where(v > 0, v, jnp.exp(jnp.minimum(v, 0.0)) - 1.0)


def _store_u(u_ref, val):
    u_ref[pl.ds(0, _N), :] = val
    u_ref[pl.ds(_N, _NP - _N), :] = jnp.zeros((_NP - _N, 128), jnp.float32)


def _tc_a_body(degp_ref, x_ref, w1_ref, dinv_ref, u1_ref):
    deg = degp_ref[0, :, 0:1] + degp_ref[1, :, 0:1] - 1.0
    dinv = lax.rsqrt(deg)
    dinv_ref[...] = dinv
    h = jnp.dot(x_ref[...], w1_ref[...], preferred_element_type=jnp.float32)
    _store_u(u1_ref, dinv * h)


_tc_a = pl.pallas_call(
    _tc_a_body,
    out_shape=(
        jax.ShapeDtypeStruct((_N, 1), jnp.float32),
        jax.ShapeDtypeStruct((_NP, 128), jnp.float32),
    ),
)


def _tc_b_body(agg_ref, u1_ref, b1_ref, dinv_ref, w2_ref, x1_ref, u2_ref):
    dinv = dinv_ref[...]
    u1 = u1_ref[pl.ds(0, _N), :]
    x1 = _elu(dinv * (agg_ref[0] + agg_ref[1] - u1) + b1_ref[...])
    x1_ref[...] = x1
    h2 = jnp.dot(x1, w2_ref[...], preferred_element_type=jnp.float32)
    _store_u(u2_ref, dinv * h2)


_tc_b = pl.pallas_call(
    _tc_b_body,
    out_shape=(
        jax.ShapeDtypeStruct((_N, 128), jnp.float32),
        jax.ShapeDtypeStruct((_NP, 128), jnp.float32),
    ),
)


def _tc_c_body(agg_ref, u2_ref, b2_ref, dinv_ref, x1_ref, wres_ref, uc_ref):
    dinv = dinv_ref[...]
    u2 = u2_ref[pl.ds(0, _N), :]
    x2 = _elu(dinv * (agg_ref[0] + agg_ref[1] - u2) + b2_ref[...])
    xc = x2 + wres_ref[0, 0] * x1_ref[...]
    _store_u(uc_ref, dinv * xc)


_tc_c = pl.pallas_call(
    _tc_c_body,
    out_shape=jax.ShapeDtypeStruct((_NP, 128), jnp.float32),
)


def _tc_d_body(agg_ref, uc_ref, dinv_ref, wmu_ref, bmu_ref, wls_ref, bls_ref,
               mu_ref, ls_ref):
    dinv = dinv_ref[...]
    s = dinv * (agg_ref[0] + agg_ref[1] - uc_ref[pl.ds(0, _N), :])
    mu_ref[...] = jnp.dot(s, wmu_ref[...],
                          preferred_element_type=jnp.float32) + bmu_ref[...]
    ls_ref[...] = jnp.dot(s, wls_ref[...],
                          preferred_element_type=jnp.float32) + bls_ref[...]


_tc_d = pl.pallas_call(
    _tc_d_body,
    out_shape=(
        jax.ShapeDtypeStruct((_N, 64), jnp.float32),
        jax.ShapeDtypeStruct((_N, 64), jnp.float32),
    ),
)


def kernel(x, edge_index, W1, b1, W2, b2, W_mu, b_mu, W_ls, b_ls, w_res):
    pad = jnp.full((_NW, _EPW_PAD - _EPW), _N, jnp.int32)
    src = jnp.concatenate(
        [edge_index[0].astype(jnp.int32).reshape(_NW, _EPW), pad],
        axis=1).reshape(_NW, _NCHUNK, _CH)
    dst = jnp.concatenate(
        [edge_index[1].astype(jnp.int32).reshape(_NW, _EPW), pad],
        axis=1).reshape(_NW, _NCHUNK, _CH)
    src1 = src.reshape(-1)
    dst1 = dst.reshape(-1)
    ones16 = jnp.ones((_N, 16), jnp.float32)
    onesc = jnp.ones((_CH, 16), jnp.float32)

    degp = _deg_call(dst, ones16, onesc)
    dinv, u1 = _tc_a(degp, x, W1)

    agg1 = _scat_call(u1, src1, dst1)
    x1, u2 = _tc_b(agg1, u1, b1.reshape(1, -1), dinv, W2)

    agg2 = _scat_call(u2, src1, dst1)
    uc = _tc_c(agg2, u2, b2.reshape(1, -1), dinv, x1, w_res.reshape(1, 1))

    agg3 = _scat_call(uc, src1, dst1)
    mu, ls = _tc_d(agg3, uc, dinv, W_mu, b_mu.reshape(1, -1),
                   W_ls, b_ls.reshape(1, -1))
    return (mu, ls)


# speedup vs baseline: 1.0164x; 1.0164x over previous
import jax
import jax.numpy as jnp
from jax import lax
from jax.experimental import pallas as pl
from jax.experimental.pallas import tpu as pltpu
from jax.experimental.pallas import tpu_sc as plsc

_N = 10000
_E = 320000
_NC = 2
_NS = 16
_NW = _NC * _NS
_EPW = _E // _NW
_CH = 128
_NCHUNK = 80
_EPW_PAD = _NCHUNK * _CH
_NPAIR = _NCHUNK // 2
_BLK = 8
_NBLK = _NCHUNK // _BLK
_NP = _N + 8
_RPT = 624
_RPT_LAST = _N - (_NS - 1) * _RPT

_mesh = plsc.VectorSubcoreMesh(core_axis_name="c", subcore_axis_name="s")


def _per_tile_rows(s, fn):
    @pl.when(s < _NS - 1)
    def _():
        fn(s * _RPT, _RPT)

    @pl.when(s == _NS - 1)
    def _():
        fn((_NS - 1) * _RPT, _RPT_LAST)


def _deg_body(dstw_hbm, ones_hbm, onesc_hbm, out_hbm, dblk, ones_v, tmp, sem):
    del sem
    c = lax.axis_index("c")
    s = lax.axis_index("s")
    wid = c * _NS + s
    _per_tile_rows(s, lambda r0, nr: pltpu.sync_copy(
        ones_hbm.at[pl.ds(r0, nr)], tmp.at[pl.ds(r0, nr)]))
    pltpu.sync_copy(onesc_hbm, ones_v)
    plsc.subcore_barrier()

    def block(b, carry):
        pltpu.sync_copy(dstw_hbm.at[wid, pl.ds(b * _BLK, _BLK)], dblk)
        for k in range(_BLK):
            pltpu.sync_copy(ones_v, tmp.at[dblk.at[k]], add=True)
        return carry

    lax.fori_loop(0, _NBLK, block, 0)
    plsc.subcore_barrier()
    _per_tile_rows(s, lambda r0, nr: pltpu.sync_copy(
        tmp.at[pl.ds(r0, nr)], out_hbm.at[c, pl.ds(r0, nr)]))


_deg_call = pl.kernel(
    _deg_body,
    out_type=jax.ShapeDtypeStruct((_NC, _N, 16), jnp.float32),
    mesh=_mesh,
    scratch_types=[
        pltpu.VMEM((_BLK, _CH), jnp.int32),
        pltpu.VMEM((_CH, 16), jnp.float32),
        pltpu.VMEM_SHARED((_NP, 16), jnp.float32),
        pltpu.SemaphoreType.DMA,
    ],
)


_SCH = 80
_SNCH = _EPW_PAD // _SCH
_SQ = _SNCH // 2


def _scat_body(u_hbm, src_hbm, dst_hbm, out_hbm,
               srcv, dstv0, dstv1, rows0, rows1, tmp, semg, sems0, sems1):
    c = lax.axis_index("c")
    s = lax.axis_index("s")
    wid = c * _NS + s
    _per_tile_rows(s, lambda r0, nr: pltpu.sync_copy(
        u_hbm.at[pl.ds(r0, nr)], tmp.at[pl.ds(r0, nr)]))
    plsc.subcore_barrier()
    base = wid * _EPW_PAD

    def drain_scat(rows, sems):
        pltpu.make_async_copy(u_hbm.at[pl.ds(0, _SCH)],
                              tmp.at[pl.ds(0, _SCH)], sems).wait()

    def chunk(cc, dstv, rows, sems, first):
        sl = pl.ds(base + cc * _SCH, _SCH)
        pltpu.sync_copy(src_hbm.at[sl], srcv)
        pltpu.sync_copy(dst_hbm.at[sl], dstv)
        pltpu.async_copy(u_hbm.at[srcv], rows, semg).wait()
        if not first:
            drain_scat(rows, sems)
        pltpu.async_copy(rows, tmp.at[dstv], sems, add=True)

    chunk(0, dstv0, rows0, sems0, True)
    chunk(1, dstv1, rows1, sems1, True)

    def pair(p, carry):
        chunk(2 * p, dstv0, rows0, sems0, False)
        chunk(2 * p + 1, dstv1, rows1, sems1, False)
        return carry

    lax.fori_loop(1, _SQ, pair, 0)
    drain_scat(rows0, sems0)
    drain_scat(rows1, sems1)
    plsc.subcore_barrier()
    _per_tile_rows(s, lambda r0, nr: pltpu.sync_copy(
        tmp.at[pl.ds(r0, nr)], out_hbm.at[c, pl.ds(r0, nr)]))


_scat_call = pl.kernel(
    _scat_body,
    out_type=jax.ShapeDtypeStruct((_NC, _N, 128), jnp.float32),
    mesh=_mesh,
    scratch_types=[
        pltpu.VMEM((_SCH,), jnp.int32),
        pltpu.VMEM((_SCH,), jnp.int32),
        pltpu.VMEM((_SCH,), jnp.int32),
        pltpu.VMEM((_SCH, 128), jnp.float32),
        pltpu.VMEM((_SCH, 128), jnp.float32),
        pltpu.VMEM_SHARED((_NP, 128), jnp.float32),
        pltpu.SemaphoreType.DMA,
        pltpu.SemaphoreType.DMA,
        pltpu.SemaphoreType.DMA,
    ],
)


def _elu(v):
    return jnp.where(v > 0, v, jnp.exp(jnp.minimum(v, 0.0)) - 1.0)


def _store_u(u_ref, val):
    u_ref[pl.ds(0, _N), :] = val
    u_ref[pl.ds(_N, _NP - _N), :] = jnp.zeros((_NP - _N, 128), jnp.float32)


def _tc_a_body(degp_ref, x_ref, w1_ref, dinv_ref, u1_ref):
    deg = degp_ref[0, :, 0:1] + degp_ref[1, :, 0:1] - 1.0
    dinv = lax.rsqrt(deg)
    dinv_ref[...] = dinv
    h = jnp.dot(x_ref[...], w1_ref[...], preferred_element_type=jnp.float32)
    _store_u(u1_ref, dinv * h)


_tc_a = pl.pallas_call(
    _tc_a_body,
    out_shape=(
        jax.ShapeDtypeStruct((_N, 1), jnp.float32),
        jax.ShapeDtypeStruct((_NP, 128), jnp.float32),
    ),
)


def _tc_b_body(agg_ref, u1_ref, b1_ref, dinv_ref, w2_ref, x1_ref, u2_ref):
    dinv = dinv_ref[...]
    u1 = u1_ref[pl.ds(0, _N), :]
    x1 = _elu(dinv * (agg_ref[0] + agg_ref[1] - u1) + b1_ref[...])
    x1_ref[...] = x1
    h2 = jnp.dot(x1, w2_ref[...], preferred_element_type=jnp.float32)
    _store_u(u2_ref, dinv * h2)


_tc_b = pl.pallas_call(
    _tc_b_body,
    out_shape=(
        jax.ShapeDtypeStruct((_N, 128), jnp.float32),
        jax.ShapeDtypeStruct((_NP, 128), jnp.float32),
    ),
)


def _tc_c_body(agg_ref, u2_ref, b2_ref, dinv_ref, x1_ref, wres_ref, uc_ref):
    dinv = dinv_ref[...]
    u2 = u2_ref[pl.ds(0, _N), :]
    x2 = _elu(dinv * (agg_ref[0] + agg_ref[1] - u2) + b2_ref[...])
    xc = x2 + wres_ref[0, 0] * x1_ref[...]
    _store_u(uc_ref, dinv * xc)


_tc_c = pl.pallas_call(
    _tc_c_body,
    out_shape=jax.ShapeDtypeStruct((_NP, 128), jnp.float32),
)


def _tc_d_body(agg_ref, uc_ref, dinv_ref, wmu_ref, bmu_ref, wls_ref, bls_ref,
               mu_ref, ls_ref):
    dinv = dinv_ref[...]
    s = dinv * (agg_ref[0] + agg_ref[1] - uc_ref[pl.ds(0, _N), :])
    mu_ref[...] = jnp.dot(s, wmu_ref[...],
                          preferred_element_type=jnp.float32) + bmu_ref[...]
    ls_ref[...] = jnp.dot(s, wls_ref[...],
                          preferred_element_type=jnp.float32) + bls_ref[...]


_tc_d = pl.pallas_call(
    _tc_d_body,
    out_shape=(
        jax.ShapeDtypeStruct((_N, 64), jnp.float32),
        jax.ShapeDtypeStruct((_N, 64), jnp.float32),
    ),
)


def kernel(x, edge_index, W1, b1, W2, b2, W_mu, b_mu, W_ls, b_ls, w_res):
    pad = jnp.full((_NW, _EPW_PAD - _EPW), _N, jnp.int32)
    src = jnp.concatenate(
        [edge_index[0].astype(jnp.int32).reshape(_NW, _EPW), pad],
        axis=1).reshape(_NW, _NCHUNK, _CH)
    dst = jnp.concatenate(
        [edge_index[1].astype(jnp.int32).reshape(_NW, _EPW), pad],
        axis=1).reshape(_NW, _NCHUNK, _CH)
    src1 = src.reshape(-1)
    dst1 = dst.reshape(-1)
    ones16 = jnp.ones((_N, 16), jnp.float32)
    onesc = jnp.ones((_CH, 16), jnp.float32)

    degp = _deg_call(dst, ones16, onesc)
    dinv, u1 = _tc_a(degp, x, W1)

    agg1 = _scat_call(u1, src1, dst1)
    x1, u2 = _tc_b(agg1, u1, b1.reshape(1, -1), dinv, W2)

    agg2 = _scat_call(u2, src1, dst1)
    uc = _tc_c(agg2, u2, b2.reshape(1, -1), dinv, x1, w_res.reshape(1, 1))

    agg3 = _scat_call(uc, src1, dst1)
    mu, ls = _tc_d(agg3, uc, dinv, W_mu, b_mu.reshape(1, -1),
                   W_ls, b_ls.reshape(1, -1))
    return (mu, ls)


# speedup vs baseline: 1.6747x; 1.6476x over previous
import jax
import jax.numpy as jnp
from jax import lax
from jax.experimental import pallas as pl
from jax.experimental.pallas import tpu as pltpu
from jax.experimental.pallas import tpu_sc as plsc

_N = 10000
_E = 320000
_NC = 2
_NS = 16
_NW = _NC * _NS
_EPW = _E // _NW
_CH = 128
_NCHUNK = 80
_EPW_PAD = _NCHUNK * _CH
_NPAIR = _NCHUNK // 2
_BLK = 8
_NBLK = _NCHUNK // _BLK
_NP = _N + 8
_RPT = 624
_RPT_LAST = _N - (_NS - 1) * _RPT

_mesh = plsc.VectorSubcoreMesh(core_axis_name="c", subcore_axis_name="s")


def _per_tile_rows(s, fn):
    @pl.when(s < _NS - 1)
    def _():
        fn(s * _RPT, _RPT)

    @pl.when(s == _NS - 1)
    def _():
        fn((_NS - 1) * _RPT, _RPT_LAST)


def _deg_body(dstw_hbm, ones_hbm, onesc_hbm, out_hbm, dblk, ones_v, tmp, sem):
    del sem
    c = lax.axis_index("c")
    s = lax.axis_index("s")
    wid = c * _NS + s
    _per_tile_rows(s, lambda r0, nr: pltpu.sync_copy(
        ones_hbm.at[pl.ds(r0, nr)], tmp.at[pl.ds(r0, nr)]))
    pltpu.sync_copy(onesc_hbm, ones_v)
    plsc.subcore_barrier()

    def block(b, carry):
        pltpu.sync_copy(dstw_hbm.at[wid, pl.ds(b * _BLK, _BLK)], dblk)
        for k in range(_BLK):
            pltpu.sync_copy(ones_v, tmp.at[dblk.at[k]], add=True)
        return carry

    lax.fori_loop(0, _NBLK, block, 0)
    plsc.subcore_barrier()
    _per_tile_rows(s, lambda r0, nr: pltpu.sync_copy(
        tmp.at[pl.ds(r0, nr)], out_hbm.at[c, pl.ds(r0, nr)]))


_deg_call = pl.kernel(
    _deg_body,
    out_type=jax.ShapeDtypeStruct((_NC, _N, 16), jnp.float32),
    mesh=_mesh,
    scratch_types=[
        pltpu.VMEM((_BLK, _CH), jnp.int32),
        pltpu.VMEM((_CH, 16), jnp.float32),
        pltpu.VMEM_SHARED((_NP, 16), jnp.float32),
        pltpu.SemaphoreType.DMA,
    ],
)


_SCH = 80
_SNCH = _EPW // _SCH


def _scat_body(u_hbm, src_hbm, dst_hbm, out_hbm, src_v, dst_v, rows_v,
               tmp, sem):
    c = lax.axis_index("c")
    s = lax.axis_index("s")
    wid = c * _NS + s
    _per_tile_rows(s, lambda r0, nr: pltpu.sync_copy(
        u_hbm.at[pl.ds(r0, nr)], tmp.at[pl.ds(r0, nr)]))
    plsc.subcore_barrier()
    base = wid * _EPW_PAD

    def chunk(i, carry):
        sl = pl.ds(base + i * _SCH, _SCH)
        pltpu.sync_copy(src_hbm.at[sl], src_v)
        pltpu.sync_copy(dst_hbm.at[sl], dst_v)
        pltpu.async_copy(u_hbm.at[src_v], rows_v, sem).wait()
        pltpu.sync_copy(rows_v, tmp.at[dst_v], add=True)
        return carry

    lax.fori_loop(0, _SNCH, chunk, 0)
    plsc.subcore_barrier()
    _per_tile_rows(s, lambda r0, nr: pltpu.sync_copy(
        tmp.at[pl.ds(r0, nr)], out_hbm.at[c, pl.ds(r0, nr)]))


_scat_call = pl.kernel(
    _scat_body,
    out_type=jax.ShapeDtypeStruct((_NC, _N, 128), jnp.float32),
    mesh=_mesh,
    scratch_types=[
        pltpu.VMEM((_SCH,), jnp.int32),
        pltpu.VMEM((_SCH,), jnp.int32),
        pltpu.VMEM((_SCH, 128), jnp.float32),
        pltpu.VMEM_SHARED((_NP, 128), jnp.float32),
        pltpu.SemaphoreType.DMA,
    ],
)


def _elu(v):
    return jnp.where(v > 0, v, jnp.exp(jnp.minimum(v, 0.0)) - 1.0)


def _store_u(u_ref, val):
    u_ref[pl.ds(0, _N), :] = val
    u_ref[pl.ds(_N, _NP - _N), :] = jnp.zeros((_NP - _N, 128), jnp.float32)


def _tc_a_body(degp_ref, x_ref, w1_ref, dinv_ref, u1_ref):
    deg = degp_ref[0, :, 0:1] + degp_ref[1, :, 0:1] - 1.0
    dinv = lax.rsqrt(deg)
    dinv_ref[...] = dinv
    h = jnp.dot(x_ref[...], w1_ref[...], preferred_element_type=jnp.float32)
    _store_u(u1_ref, dinv * h)


_tc_a = pl.pallas_call(
    _tc_a_body,
    out_shape=(
        jax.ShapeDtypeStruct((_N, 1), jnp.float32),
        jax.ShapeDtypeStruct((_NP, 128), jnp.float32),
    ),
)


def _tc_b_body(agg_ref, u1_ref, b1_ref, dinv_ref, w2_ref, x1_ref, u2_ref):
    dinv = dinv_ref[...]
    u1 = u1_ref[pl.ds(0, _N), :]
    x1 = _elu(dinv * (agg_ref[0] + agg_ref[1] - u1) + b1_ref[...])
    x1_ref[...] = x1
    h2 = jnp.dot(x1, w2_ref[...], preferred_element_type=jnp.float32)
    _store_u(u2_ref, dinv * h2)


_tc_b = pl.pallas_call(
    _tc_b_body,
    out_shape=(
        jax.ShapeDtypeStruct((_N, 128), jnp.float32),
        jax.ShapeDtypeStruct((_NP, 128), jnp.float32),
    ),
)


def _tc_c_body(agg_ref, u2_ref, b2_ref, dinv_ref, x1_ref, wres_ref, uc_ref):
    dinv = dinv_ref[...]
    u2 = u2_ref[pl.ds(0, _N), :]
    x2 = _elu(dinv * (agg_ref[0] + agg_ref[1] - u2) + b2_ref[...])
    xc = x2 + wres_ref[0, 0] * x1_ref[...]
    _store_u(uc_ref, dinv * xc)


_tc_c = pl.pallas_call(
    _tc_c_body,
    out_shape=jax.ShapeDtypeStruct((_NP, 128), jnp.float32),
)


def _tc_d_body(agg_ref, uc_ref, dinv_ref, wmu_ref, bmu_ref, wls_ref, bls_ref,
               mu_ref, ls_ref):
    dinv = dinv_ref[...]
    s = dinv * (agg_ref[0] + agg_ref[1] - uc_ref[pl.ds(0, _N), :])
    mu_ref[...] = jnp.dot(s, wmu_ref[...],
                          preferred_element_type=jnp.float32) + bmu_ref[...]
    ls_ref[...] = jnp.dot(s, wls_ref[...],
                          preferred_element_type=jnp.float32) + bls_ref[...]


_tc_d = pl.pallas_call(
    _tc_d_body,
    out_shape=(
        jax.ShapeDtypeStruct((_N, 64), jnp.float32),
        jax.ShapeDtypeStruct((_N, 64), jnp.float32),
    ),
)


def kernel(x, edge_index, W1, b1, W2, b2, W_mu, b_mu, W_ls, b_ls, w_res):
    pad = jnp.full((_NW, _EPW_PAD - _EPW), _N, jnp.int32)
    src = jnp.concatenate(
        [edge_index[0].astype(jnp.int32).reshape(_NW, _EPW), pad],
        axis=1).reshape(_NW, _NCHUNK, _CH)
    dst = jnp.concatenate(
        [edge_index[1].astype(jnp.int32).reshape(_NW, _EPW), pad],
        axis=1).reshape(_NW, _NCHUNK, _CH)
    src1 = src.reshape(-1)
    dst1 = dst.reshape(-1)
    ones16 = jnp.ones((_N, 16), jnp.float32)
    onesc = jnp.ones((_CH, 16), jnp.float32)

    degp = _deg_call(dst, ones16, onesc)
    dinv, u1 = _tc_a(degp, x, W1)

    agg1 = _scat_call(u1, src1, dst1)
    x1, u2 = _tc_b(agg1, u1, b1.reshape(1, -1), dinv, W2)

    agg2 = _scat_call(u2, src1, dst1)
    uc = _tc_c(agg2, u2, b2.reshape(1, -1), dinv, x1, w_res.reshape(1, 1))

    agg3 = _scat_call(uc, src1, dst1)
    mu, ls = _tc_d(agg3, uc, dinv, W_mu, b_mu.reshape(1, -1),
                   W_ls, b_ls.reshape(1, -1))
    return (mu, ls)
